# Initial kernel scaffold; baseline (speedup 1.0000x reference)
import jax
import jax.numpy as jnp
from jax.experimental import pallas as pl


def kernel(xyz, normals):
    raise NotImplementedError("write your pallas kernel here")


# speedup vs baseline: 14.1698x; 14.1698x over previous
import functools

import jax
import jax.numpy as jnp
from jax import lax
from jax.experimental import pallas as pl
from jax.experimental.pallas import tpu as pltpu
from jax.experimental.pallas import tpu_sc as plsc

_ROWS_A = 64
_K = 16
_ROWS_C = 400


def _topk_body(a_ref, b_ref, sq_ref, o_ref):
    R = a_ref.shape[0]
    W = b_ref.shape[1]
    i = pl.program_id(0)
    def r16(v):
        return v.astype(jnp.bfloat16).astype(jnp.float32)

    xi = r16(a_ref[:, 0:1])
    yi = r16(a_ref[:, 1:2])
    zi = r16(a_ref[:, 2:3])
    xj = r16(b_ref[0:1, :])
    yj = r16(b_ref[1:2, :])
    zj = r16(b_ref[2:3, :])
    sqj = b_ref[3:4, :]
    term = xi * xj + yi * yj + zi * zj
    d = jnp.maximum((sq_ref[...] + sqj) - 2.0 * term, 0.0)
    rows = i * R + lax.broadcasted_iota(jnp.int32, (R, 1), 0)
    cols = lax.broadcasted_iota(jnp.int32, (R, W), 1)
    inf = jnp.float32(jnp.inf)
    d = jnp.where(cols == rows, inf, d)
    m = jnp.min(d, axis=1, keepdims=True)
    for t in range(_K):
        idx = jnp.min(jnp.where(d <= m, cols, W), axis=1, keepdims=True)
        o_ref[:, t:t + 1] = idx
        d = jnp.where(cols == idx, inf, d)
        if t < _K - 1:
            m = jnp.min(d, axis=1, keepdims=True)


def _feat_body(nx_ref, ny_ref, nz_ref, nrm_ref, o_ref):
    k = jnp.float32(_K)
    nx, ny, nz = nx_ref[...], ny_ref[...], nz_ref[...]
    mx = jnp.sum(nx, axis=1, keepdims=True) / k
    my = jnp.sum(ny, axis=1, keepdims=True) / k
    mz = jnp.sum(nz, axis=1, keepdims=True) / k
    cx, cy, cz = nx - mx, ny - my, nz - mz
    denom = jnp.float32(_K - 1)
    a00 = jnp.sum(cx * cx, axis=1, keepdims=True) / denom
    a11 = jnp.sum(cy * cy, axis=1, keepdims=True) / denom
    a22 = jnp.sum(cz * cz, axis=1, keepdims=True) / denom
    a01 = jnp.sum(cx * cy, axis=1, keepdims=True) / denom
    a02 = jnp.sum(cx * cz, axis=1, keepdims=True) / denom
    a12 = jnp.sum(cy * cz, axis=1, keepdims=True) / denom

    q = (a00 + a11 + a22) / 3.0
    p1 = a01 * a01 + a02 * a02 + a12 * a12
    p2 = (a00 - q) ** 2 + (a11 - q) ** 2 + (a22 - q) ** 2 + 2.0 * p1
    p = jnp.sqrt(jnp.maximum(p2, 0.0) / 6.0)
    ps = jnp.maximum(p, jnp.float32(1e-20))
    b00, b11, b22 = (a00 - q) / ps, (a11 - q) / ps, (a22 - q) / ps
    b01, b02, b12 = a01 / ps, a02 / ps, a12 / ps
    detb = (b00 * (b11 * b22 - b12 * b12)
            - b01 * (b01 * b22 - b12 * b02)
            + b02 * (b01 * b12 - b11 * b02))
    r = jnp.clip(detb * 0.5, -1.0, 1.0)
    mu0 = jnp.full_like(r, 2.0)
    mu2 = jnp.full_like(r, -2.0)
    for _ in range(20):
        g = mu0 * mu0 * mu0 - 3.0 * mu0 - 2.0 * r
        gp = jnp.maximum(3.0 * mu0 * mu0 - 3.0, jnp.float32(1e-12))
        mu0 = jnp.clip(mu0 - g / gp, 1.0, 2.0)
        g = mu2 * mu2 * mu2 - 3.0 * mu2 - 2.0 * r
        gp = jnp.maximum(3.0 * mu2 * mu2 - 3.0, jnp.float32(1e-12))
        mu2 = jnp.clip(mu2 - g / gp, -2.0, -1.0)
    w0 = q + p * mu0
    w2 = q + p * mu2
    c00, c11, c22 = a00 - w2, a11 - w2, a22 - w2
    v1x = a01 * a12 - a02 * c11
    v1y = a02 * a01 - c00 * a12
    v1z = c00 * c11 - a01 * a01
    v2x = a01 * c22 - a02 * a12
    v2y = a02 * a02 - c00 * c22
    v2z = c00 * a12 - a01 * a02
    v3x = c11 * c22 - a12 * a12
    v3y = a12 * a02 - a01 * c22
    v3z = a01 * a12 - c11 * a02
    n1 = v1x * v1x + v1y * v1y + v1z * v1z
    n2 = v2x * v2x + v2y * v2y + v2z * v2z
    n3 = v3x * v3x + v3y * v3y + v3z * v3z
    use1 = jnp.logical_and(n1 >= n2, n1 >= n3)
    use2 = n2 >= n3
    vx = jnp.where(use1, v1x, jnp.where(use2, v2x, v3x))
    vy = jnp.where(use1, v1y, jnp.where(use2, v2y, v3y))
    vz = jnp.where(use1, v1z, jnp.where(use2, v2z, v3z))
    nv = jnp.sqrt(vx * vx + vy * vy + vz * vz)
    nv = jnp.maximum(nv, jnp.float32(1e-20))
    vx, vy, vz = vx / nv, vy / nv, vz / nv

    nrm = nrm_ref[...]
    n0, n1c, n2c = nrm[:, 0:1], nrm[:, 1:2], nrm[:, 2:3]
    dot = vx * n0 + vy * n1c + vz * n2c
    s = jnp.sign(dot)
    fx, fy, fz = vx * s, vy * s, vz * s
    consistency = jnp.abs(dot)
    total_var = jnp.maximum(a00 + a11 + a22, jnp.float32(1e-8))
    curvature = w2 / total_var
    perp = jnp.abs(cx * fx + cy * fy + cz * fz)
    rough = jnp.sum(perp, axis=1, keepdims=True) / k
    o_ref[...] = jnp.concatenate([consistency, nrm, curvature, rough], axis=1)


def _topk_indices(xyz):
    n = xyz.shape[0]
    npad = ((n + _ROWS_A - 1) // _ROWS_A) * _ROWS_A
    pad = npad - n
    xyz_p = jnp.pad(xyz, ((0, pad), (0, 0)), constant_values=1e9)
    sq = jnp.sum(xyz_p * xyz_p, axis=1, keepdims=True)
    zeros = jnp.zeros((npad, 4), jnp.float32)
    b_mat = jnp.concatenate([xyz_p.T, sq.T, zeros.T], axis=0)
    grid = npad // _ROWS_A
    idx = pl.pallas_call(
        _topk_body,
        grid=(grid,),
        in_specs=[
            pl.BlockSpec((_ROWS_A, 3), lambda i: (i, 0)),
            pl.BlockSpec((8, npad), lambda i: (0, 0)),
            pl.BlockSpec((_ROWS_A, 1), lambda i: (i, 0)),
        ],
        out_specs=pl.BlockSpec((_ROWS_A, _K), lambda i: (i, 0)),
        out_shape=jax.ShapeDtypeStruct((npad, _K), jnp.int32),
    )(xyz_p, b_mat, sq)
    return idx[:n]


def _sc_gather(table, idx_flat):
    info = plsc.get_sparse_core_info()
    nw = info.num_cores * info.num_subcores
    b = idx_flat.shape[0]
    quantum = nw * 128
    b_pad = ((b + quantum - 1) // quantum) * quantum
    idx_flat = jnp.pad(idx_flat, (0, b_pad - b))
    chunks = b_pad // quantum
    idx3 = idx_flat.reshape(nw, chunks, 128)
    mesh = plsc.VectorSubcoreMesh(core_axis_name="c", subcore_axis_name="s")

    @functools.partial(
        pl.kernel, mesh=mesh,
        compiler_params=pltpu.CompilerParams(use_tc_tiling_on_sc=False),
        out_type=jax.ShapeDtypeStruct((nw, chunks, 128, 16), jnp.float32),
        scratch_types=[
            pltpu.VMEM((chunks, 128), jnp.int32),
            pltpu.VMEM((128, 16), jnp.float32),
            pltpu.SemaphoreType.DMA,
        ],
    )
    def gather_k(table_hbm, idx_hbm, out_hbm, idx_v, rows_v, sem):
        wid = lax.axis_index("s") * info.num_cores + lax.axis_index("c")
        pltpu.sync_copy(idx_hbm.at[wid], idx_v)

        def body(j, carry):
            pltpu.async_copy(table_hbm.at[idx_v.at[j]], rows_v, sem).wait()
            pltpu.sync_copy(rows_v, out_hbm.at[wid, j])
            return carry

        lax.fori_loop(0, chunks, body, 0)

    out = gather_k(table, idx3)
    return out.reshape(b_pad, 16)[:b]


def kernel(xyz, normals):
    n = xyz.shape[0]
    idx = _topk_indices(xyz)
    table = jnp.pad(xyz, ((0, 0), (0, 13)))
    rows = _sc_gather(table, idx.reshape(-1))
    nb = rows.reshape(n, _K, 16)
    nx, ny, nz = nb[:, :, 0], nb[:, :, 1], nb[:, :, 2]
    grid = n // _ROWS_C
    out = pl.pallas_call(
        _feat_body,
        grid=(grid,),
        in_specs=[
            pl.BlockSpec((_ROWS_C, _K), lambda i: (i, 0)),
            pl.BlockSpec((_ROWS_C, _K), lambda i: (i, 0)),
            pl.BlockSpec((_ROWS_C, _K), lambda i: (i, 0)),
            pl.BlockSpec((_ROWS_C, 3), lambda i: (i, 0)),
        ],
        out_specs=pl.BlockSpec((_ROWS_C, 6), lambda i: (i, 0)),
        out_shape=jax.ShapeDtypeStruct((n, 6), jnp.float32),
    )(nx, ny, nz, normals)
    return out


# speedup vs baseline: 20.8742x; 1.4731x over previous
import functools

import jax
import jax.numpy as jnp
import numpy as np
from jax import lax
from jax.experimental import pallas as pl
from jax.experimental.pallas import tpu as pltpu
from jax.experimental.pallas import tpu_sc as plsc
from jax.sharding import Mesh, PartitionSpec as P
from jax.experimental.shard_map import shard_map

_ROWS_A = 64
_K = 16
_ROWS_C = 200


def _topk_body(a_ref, b_ref, sq_ref, rows_ref, o_ref):
    R = a_ref.shape[0]
    W = b_ref.shape[1]
    def r16(v):
        return v.astype(jnp.bfloat16).astype(jnp.float32)

    xi = r16(a_ref[:, 0:1])
    yi = r16(a_ref[:, 1:2])
    zi = r16(a_ref[:, 2:3])
    xj = r16(b_ref[0:1, :])
    yj = r16(b_ref[1:2, :])
    zj = r16(b_ref[2:3, :])
    sqj = b_ref[3:4, :]
    term = xi * xj + yi * yj + zi * zj
    d = jnp.maximum((sq_ref[...] + sqj) - 2.0 * term, 0.0)
    rows = rows_ref[...]
    cols = lax.broadcasted_iota(jnp.int32, (R, W), 1)
    inf = jnp.float32(jnp.inf)
    d = jnp.where(cols == rows, inf, d)
    m = jnp.min(d, axis=1, keepdims=True)
    for t in range(_K):
        idx = jnp.min(jnp.where(d <= m, cols, W), axis=1, keepdims=True)
        o_ref[:, t:t + 1] = idx
        d = jnp.where(cols == idx, inf, d)
        if t < _K - 1:
            m = jnp.min(d, axis=1, keepdims=True)


def _feat_body(nx_ref, ny_ref, nz_ref, nrm_ref, o_ref):
    k = jnp.float32(_K)
    nx, ny, nz = nx_ref[...], ny_ref[...], nz_ref[...]
    mx = jnp.sum(nx, axis=1, keepdims=True) / k
    my = jnp.sum(ny, axis=1, keepdims=True) / k
    mz = jnp.sum(nz, axis=1, keepdims=True) / k
    cx, cy, cz = nx - mx, ny - my, nz - mz
    denom = jnp.float32(_K - 1)
    a00 = jnp.sum(cx * cx, axis=1, keepdims=True) / denom
    a11 = jnp.sum(cy * cy, axis=1, keepdims=True) / denom
    a22 = jnp.sum(cz * cz, axis=1, keepdims=True) / denom
    a01 = jnp.sum(cx * cy, axis=1, keepdims=True) / denom
    a02 = jnp.sum(cx * cz, axis=1, keepdims=True) / denom
    a12 = jnp.sum(cy * cz, axis=1, keepdims=True) / denom

    q = (a00 + a11 + a22) / 3.0
    p1 = a01 * a01 + a02 * a02 + a12 * a12
    p2 = (a00 - q) ** 2 + (a11 - q) ** 2 + (a22 - q) ** 2 + 2.0 * p1
    p = jnp.sqrt(jnp.maximum(p2, 0.0) / 6.0)
    ps = jnp.maximum(p, jnp.float32(1e-20))
    b00, b11, b22 = (a00 - q) / ps, (a11 - q) / ps, (a22 - q) / ps
    b01, b02, b12 = a01 / ps, a02 / ps, a12 / ps
    detb = (b00 * (b11 * b22 - b12 * b12)
            - b01 * (b01 * b22 - b12 * b02)
            + b02 * (b01 * b12 - b11 * b02))
    r = jnp.clip(detb * 0.5, -1.0, 1.0)
    mu0 = jnp.full_like(r, 2.0)
    mu2 = jnp.full_like(r, -2.0)
    for _ in range(20):
        g = mu0 * mu0 * mu0 - 3.0 * mu0 - 2.0 * r
        gp = jnp.maximum(3.0 * mu0 * mu0 - 3.0, jnp.float32(1e-12))
        mu0 = jnp.clip(mu0 - g / gp, 1.0, 2.0)
        g = mu2 * mu2 * mu2 - 3.0 * mu2 - 2.0 * r
        gp = jnp.maximum(3.0 * mu2 * mu2 - 3.0, jnp.float32(1e-12))
        mu2 = jnp.clip(mu2 - g / gp, -2.0, -1.0)
    w0 = q + p * mu0
    w2 = q + p * mu2
    c00, c11, c22 = a00 - w2, a11 - w2, a22 - w2
    v1x = a01 * a12 - a02 * c11
    v1y = a02 * a01 - c00 * a12
    v1z = c00 * c11 - a01 * a01
    v2x = a01 * c22 - a02 * a12
    v2y = a02 * a02 - c00 * c22
    v2z = c00 * a12 - a01 * a02
    v3x = c11 * c22 - a12 * a12
    v3y = a12 * a02 - a01 * c22
    v3z = a01 * a12 - c11 * a02
    n1 = v1x * v1x + v1y * v1y + v1z * v1z
    n2 = v2x * v2x + v2y * v2y + v2z * v2z
    n3 = v3x * v3x + v3y * v3y + v3z * v3z
    use1 = jnp.logical_and(n1 >= n2, n1 >= n3)
    use2 = n2 >= n3
    vx = jnp.where(use1, v1x, jnp.where(use2, v2x, v3x))
    vy = jnp.where(use1, v1y, jnp.where(use2, v2y, v3y))
    vz = jnp.where(use1, v1z, jnp.where(use2, v2z, v3z))
    nv = jnp.sqrt(vx * vx + vy * vy + vz * vz)
    nv = jnp.maximum(nv, jnp.float32(1e-20))
    vx, vy, vz = vx / nv, vy / nv, vz / nv

    nrm = nrm_ref[...]
    n0, n1c, n2c = nrm[:, 0:1], nrm[:, 1:2], nrm[:, 2:3]
    dot = vx * n0 + vy * n1c + vz * n2c
    s = jnp.sign(dot)
    fx, fy, fz = vx * s, vy * s, vz * s
    consistency = jnp.abs(dot)
    total_var = jnp.maximum(a00 + a11 + a22, jnp.float32(1e-8))
    curvature = w2 / total_var
    perp = jnp.abs(cx * fx + cy * fy + cz * fz)
    rough = jnp.sum(perp, axis=1, keepdims=True) / k
    o_ref[...] = jnp.concatenate([consistency, nrm, curvature, rough], axis=1)


def _topk_call(xyz_blk, b_mat, sq_blk, rows_blk):
    rloc = xyz_blk.shape[0]
    npad = b_mat.shape[1]
    grid = rloc // _ROWS_A
    return pl.pallas_call(
        _topk_body,
        grid=(grid,),
        in_specs=[
            pl.BlockSpec((_ROWS_A, 3), lambda i: (i, 0)),
            pl.BlockSpec((8, npad), lambda i: (0, 0)),
            pl.BlockSpec((_ROWS_A, 1), lambda i: (i, 0)),
            pl.BlockSpec((_ROWS_A, 1), lambda i: (i, 0)),
        ],
        out_specs=pl.BlockSpec((_ROWS_A, _K), lambda i: (i, 0)),
        out_shape=jax.ShapeDtypeStruct((rloc, _K), jnp.int32),
    )(xyz_blk, b_mat, sq_blk, rows_blk)


def _sc_gather(table, idx_flat):
    info = plsc.get_sparse_core_info()
    nw = info.num_cores * info.num_subcores
    b = idx_flat.shape[0]
    quantum = nw * 128
    b_pad = ((b + quantum - 1) // quantum) * quantum
    idx_flat = jnp.pad(idx_flat, (0, b_pad - b))
    chunks = b_pad // quantum
    idx3 = idx_flat.reshape(nw, chunks, 128)
    mesh = plsc.VectorSubcoreMesh(core_axis_name="c", subcore_axis_name="s")

    @functools.partial(
        pl.kernel, mesh=mesh,
        compiler_params=pltpu.CompilerParams(use_tc_tiling_on_sc=False),
        out_type=jax.ShapeDtypeStruct((nw, chunks, 128, 16), jnp.float32),
        scratch_types=[
            pltpu.VMEM((chunks, 128), jnp.int32),
            pltpu.VMEM((128, 16), jnp.float32),
            pltpu.SemaphoreType.DMA,
        ],
    )
    def gather_k(table_hbm, idx_hbm, out_hbm, idx_v, rows_v, sem):
        wid = lax.axis_index("s") * info.num_cores + lax.axis_index("c")
        pltpu.sync_copy(idx_hbm.at[wid], idx_v)

        def body(j, carry):
            pltpu.async_copy(table_hbm.at[idx_v.at[j]], rows_v, sem).wait()
            pltpu.sync_copy(rows_v, out_hbm.at[wid, j])
            return carry

        lax.fori_loop(0, chunks, body, 0)

    out = gather_k(table, idx3)
    return out.reshape(b_pad, 16)[:b]


def _feat_call(nx, ny, nz, normals):
    nloc = nx.shape[0]
    grid = nloc // _ROWS_C
    return pl.pallas_call(
        _feat_body,
        grid=(grid,),
        in_specs=[
            pl.BlockSpec((_ROWS_C, _K), lambda i: (i, 0)),
            pl.BlockSpec((_ROWS_C, _K), lambda i: (i, 0)),
            pl.BlockSpec((_ROWS_C, _K), lambda i: (i, 0)),
            pl.BlockSpec((_ROWS_C, 3), lambda i: (i, 0)),
        ],
        out_specs=pl.BlockSpec((_ROWS_C, 6), lambda i: (i, 0)),
        out_shape=jax.ShapeDtypeStruct((nloc, 6), jnp.float32),
    )(nx, ny, nz, normals)


def kernel(xyz, normals):
    n = xyz.shape[0]
    devs = jax.devices()
    ndev = len(devs)
    mesh = Mesh(np.array(devs), ("x",))
    repl = P(None, None)
    shard = P("x", None)

    quantum = ndev * _ROWS_A
    npad = ((n + quantum - 1) // quantum) * quantum
    xyz_p = jnp.pad(xyz, ((0, npad - n), (0, 0)), constant_values=1e9)
    sq = jnp.sum(xyz_p * xyz_p, axis=1, keepdims=True)
    zeros = jnp.zeros((npad, 4), jnp.float32)
    b_mat = jnp.concatenate([xyz_p.T, sq.T, zeros.T], axis=0)
    rows_g = jnp.arange(npad, dtype=jnp.int32)[:, None]

    topk = shard_map(_topk_call, mesh=mesh,
                     in_specs=(shard, repl, shard, shard),
                     out_specs=shard, check_rep=False)
    idx = topk(xyz_p, b_mat, sq, rows_g)[:n]

    table = jnp.pad(xyz, ((0, 0), (0, 13)))
    gather = shard_map(_sc_gather, mesh=mesh,
                       in_specs=(repl, P("x")), out_specs=shard,
                       check_rep=False)
    rows = gather(table, idx.reshape(-1))
    nb = rows.reshape(n, _K, 16)
    nx, ny, nz = nb[:, :, 0], nb[:, :, 1], nb[:, :, 2]

    feats = shard_map(_feat_call, mesh=mesh,
                      in_specs=(shard, shard, shard, shard),
                      out_specs=shard, check_rep=False)
    return feats(nx, ny, nz, normals)


# speedup vs baseline: 21.1094x; 1.0113x over previous
import functools

import jax
import jax.numpy as jnp
import numpy as np
from jax import lax
from jax.experimental import pallas as pl
from jax.experimental.pallas import tpu as pltpu
from jax.experimental.pallas import tpu_sc as plsc
from jax.sharding import Mesh, PartitionSpec as P
from jax.experimental.shard_map import shard_map

_ROWS_A = 64
_K = 16
_ROWS_C = 200


def _topk_body(a_ref, b_ref, sq_ref, rows_ref, o_ref):
    R = a_ref.shape[0]
    W = b_ref.shape[1]
    def r16(v):
        return v.astype(jnp.bfloat16).astype(jnp.float32)

    xi = r16(a_ref[:, 0:1])
    yi = r16(a_ref[:, 1:2])
    zi = r16(a_ref[:, 2:3])
    xj = r16(b_ref[0:1, :])
    yj = r16(b_ref[1:2, :])
    zj = r16(b_ref[2:3, :])
    sqj = b_ref[3:4, :]
    term = xi * xj + yi * yj + zi * zj
    d = jnp.maximum((sq_ref[...] + sqj) - 2.0 * term, 0.0)
    rows = rows_ref[...]
    cols = lax.broadcasted_iota(jnp.int32, (R, W), 1)
    inf = jnp.float32(jnp.inf)
    d = jnp.where(cols == rows, inf, d)
    m = jnp.min(d, axis=1, keepdims=True)
    for t in range(_K):
        idx = jnp.min(jnp.where(d <= m, cols, W), axis=1, keepdims=True)
        o_ref[:, t:t + 1] = idx
        d = jnp.where(cols == idx, inf, d)
        if t < _K - 1:
            m = jnp.min(d, axis=1, keepdims=True)


def _feat_body(nx_ref, ny_ref, nz_ref, nrm_ref, o_ref):
    k = jnp.float32(_K)
    nx, ny, nz = nx_ref[...], ny_ref[...], nz_ref[...]
    mx = jnp.sum(nx, axis=1, keepdims=True) / k
    my = jnp.sum(ny, axis=1, keepdims=True) / k
    mz = jnp.sum(nz, axis=1, keepdims=True) / k
    cx, cy, cz = nx - mx, ny - my, nz - mz
    denom = jnp.float32(_K - 1)
    a00 = jnp.sum(cx * cx, axis=1, keepdims=True) / denom
    a11 = jnp.sum(cy * cy, axis=1, keepdims=True) / denom
    a22 = jnp.sum(cz * cz, axis=1, keepdims=True) / denom
    a01 = jnp.sum(cx * cy, axis=1, keepdims=True) / denom
    a02 = jnp.sum(cx * cz, axis=1, keepdims=True) / denom
    a12 = jnp.sum(cy * cz, axis=1, keepdims=True) / denom

    q = (a00 + a11 + a22) / 3.0
    p1 = a01 * a01 + a02 * a02 + a12 * a12
    p2 = (a00 - q) ** 2 + (a11 - q) ** 2 + (a22 - q) ** 2 + 2.0 * p1
    p = jnp.sqrt(jnp.maximum(p2, 0.0) / 6.0)
    ps = jnp.maximum(p, jnp.float32(1e-20))
    b00, b11, b22 = (a00 - q) / ps, (a11 - q) / ps, (a22 - q) / ps
    b01, b02, b12 = a01 / ps, a02 / ps, a12 / ps
    detb = (b00 * (b11 * b22 - b12 * b12)
            - b01 * (b01 * b22 - b12 * b02)
            + b02 * (b01 * b12 - b11 * b02))
    r = jnp.clip(detb * 0.5, -1.0, 1.0)
    mu0 = jnp.full_like(r, 2.0)
    mu2 = jnp.full_like(r, -2.0)
    for _ in range(20):
        g = mu0 * mu0 * mu0 - 3.0 * mu0 - 2.0 * r
        gp = jnp.maximum(3.0 * mu0 * mu0 - 3.0, jnp.float32(1e-12))
        mu0 = jnp.clip(mu0 - g / gp, 1.0, 2.0)
        g = mu2 * mu2 * mu2 - 3.0 * mu2 - 2.0 * r
        gp = jnp.maximum(3.0 * mu2 * mu2 - 3.0, jnp.float32(1e-12))
        mu2 = jnp.clip(mu2 - g / gp, -2.0, -1.0)
    w0 = q + p * mu0
    w2 = q + p * mu2
    c00, c11, c22 = a00 - w2, a11 - w2, a22 - w2
    v1x = a01 * a12 - a02 * c11
    v1y = a02 * a01 - c00 * a12
    v1z = c00 * c11 - a01 * a01
    v2x = a01 * c22 - a02 * a12
    v2y = a02 * a02 - c00 * c22
    v2z = c00 * a12 - a01 * a02
    v3x = c11 * c22 - a12 * a12
    v3y = a12 * a02 - a01 * c22
    v3z = a01 * a12 - c11 * a02
    n1 = v1x * v1x + v1y * v1y + v1z * v1z
    n2 = v2x * v2x + v2y * v2y + v2z * v2z
    n3 = v3x * v3x + v3y * v3y + v3z * v3z
    use1 = jnp.logical_and(n1 >= n2, n1 >= n3)
    use2 = n2 >= n3
    vx = jnp.where(use1, v1x, jnp.where(use2, v2x, v3x))
    vy = jnp.where(use1, v1y, jnp.where(use2, v2y, v3y))
    vz = jnp.where(use1, v1z, jnp.where(use2, v2z, v3z))
    nv = jnp.sqrt(vx * vx + vy * vy + vz * vz)
    nv = jnp.maximum(nv, jnp.float32(1e-20))
    vx, vy, vz = vx / nv, vy / nv, vz / nv

    nrm = nrm_ref[...]
    n0, n1c, n2c = nrm[:, 0:1], nrm[:, 1:2], nrm[:, 2:3]
    dot = vx * n0 + vy * n1c + vz * n2c
    s = jnp.sign(dot)
    fx, fy, fz = vx * s, vy * s, vz * s
    consistency = jnp.abs(dot)
    total_var = jnp.maximum(a00 + a11 + a22, jnp.float32(1e-8))
    curvature = w2 / total_var
    perp = jnp.abs(cx * fx + cy * fy + cz * fz)
    rough = jnp.sum(perp, axis=1, keepdims=True) / k
    o_ref[...] = jnp.concatenate([consistency, nrm, curvature, rough], axis=1)


def _topk_call(xyz_blk, b_mat, sq_blk, rows_blk):
    rloc = xyz_blk.shape[0]
    npad = b_mat.shape[1]
    grid = rloc // _ROWS_A
    return pl.pallas_call(
        _topk_body,
        grid=(grid,),
        in_specs=[
            pl.BlockSpec((_ROWS_A, 3), lambda i: (i, 0)),
            pl.BlockSpec((8, npad), lambda i: (0, 0)),
            pl.BlockSpec((_ROWS_A, 1), lambda i: (i, 0)),
            pl.BlockSpec((_ROWS_A, 1), lambda i: (i, 0)),
        ],
        out_specs=pl.BlockSpec((_ROWS_A, _K), lambda i: (i, 0)),
        out_shape=jax.ShapeDtypeStruct((rloc, _K), jnp.int32),
    )(xyz_blk, b_mat, sq_blk, rows_blk)


def _sc_gather(table, idx_flat):
    info = plsc.get_sparse_core_info()
    nw = info.num_cores * info.num_subcores
    b = idx_flat.shape[0]
    quantum = nw * 128
    b_pad = ((b + quantum - 1) // quantum) * quantum
    idx_flat = jnp.pad(idx_flat, (0, b_pad - b))
    chunks = b_pad // quantum
    idx3 = idx_flat.reshape(nw, chunks, 128)
    mesh = plsc.VectorSubcoreMesh(core_axis_name="c", subcore_axis_name="s")

    @functools.partial(
        pl.kernel, mesh=mesh,
        compiler_params=pltpu.CompilerParams(use_tc_tiling_on_sc=False),
        out_type=jax.ShapeDtypeStruct((nw, chunks, 128, 16), jnp.float32),
        scratch_types=[
            pltpu.VMEM((chunks, 128), jnp.int32),
            pltpu.VMEM((128, 16), jnp.float32),
            pltpu.SemaphoreType.DMA,
        ],
    )
    def gather_k(table_hbm, idx_hbm, out_hbm, idx_v, rows_v, sem):
        wid = lax.axis_index("s") * info.num_cores + lax.axis_index("c")
        pltpu.sync_copy(idx_hbm.at[wid], idx_v)

        def body(j, carry):
            pltpu.async_copy(table_hbm.at[idx_v.at[j]], rows_v, sem).wait()
            pltpu.sync_copy(rows_v, out_hbm.at[wid, j])
            return carry

        lax.fori_loop(0, chunks, body, 0)

    out = gather_k(table, idx3)
    return out.reshape(b_pad, 16)[:b]


def _feat_call(nx, ny, nz, normals):
    nloc = nx.shape[0]
    grid = nloc // _ROWS_C
    return pl.pallas_call(
        _feat_body,
        grid=(grid,),
        in_specs=[
            pl.BlockSpec((_ROWS_C, _K), lambda i: (i, 0)),
            pl.BlockSpec((_ROWS_C, _K), lambda i: (i, 0)),
            pl.BlockSpec((_ROWS_C, _K), lambda i: (i, 0)),
            pl.BlockSpec((_ROWS_C, 3), lambda i: (i, 0)),
        ],
        out_specs=pl.BlockSpec((_ROWS_C, 6), lambda i: (i, 0)),
        out_shape=jax.ShapeDtypeStruct((nloc, 6), jnp.float32),
    )(nx, ny, nz, normals)


def kernel(xyz, normals):
    n = xyz.shape[0]
    ctx = jax.sharding.get_abstract_mesh()
    if not ctx.empty and ctx.axis_names == ("x",):
        mesh = ctx
        ndev = ctx.shape["x"]
    else:
        devs = jax.devices()
        ndev = len(devs)
        mesh = Mesh(np.array(devs), ("x",))
    repl = P(None, None)
    shard = P("x", None)

    quantum = ndev * _ROWS_A
    npad = ((n + quantum - 1) // quantum) * quantum
    xyz_p = jnp.pad(xyz, ((0, npad - n), (0, 0)), constant_values=1e9)
    sq = jnp.sum(xyz_p * xyz_p, axis=1, keepdims=True)
    zeros = jnp.zeros((npad, 4), jnp.float32)
    b_mat = jnp.concatenate([xyz_p.T, sq.T, zeros.T], axis=0)
    rows_g = jnp.arange(npad, dtype=jnp.int32)[:, None]

    topk = shard_map(_topk_call, mesh=mesh,
                     in_specs=(shard, repl, shard, shard),
                     out_specs=shard, check_rep=False)
    idx = topk(xyz_p, b_mat, sq, rows_g)[:n]

    table = jnp.pad(xyz, ((0, 0), (0, 13)))
    gather = shard_map(_sc_gather, mesh=mesh,
                       in_specs=(repl, P("x")), out_specs=shard,
                       check_rep=False)
    rows = gather(table, idx.reshape(-1))
    nb = rows.reshape(n, _K, 16)
    nx, ny, nz = nb[:, :, 0], nb[:, :, 1], nb[:, :, 2]

    feats = shard_map(_feat_call, mesh=mesh,
                      in_specs=(shard, shard, shard, shard),
                      out_specs=shard, check_rep=False)
    return feats(nx, ny, nz, normals)


# speedup vs baseline: 21.6818x; 1.0271x over previous
import functools

import jax
import jax.numpy as jnp
import numpy as np
from jax import lax
from jax.experimental import pallas as pl
from jax.experimental.pallas import tpu as pltpu
from jax.experimental.pallas import tpu_sc as plsc
from jax.sharding import Mesh, PartitionSpec as P
from jax.experimental.shard_map import shard_map

_ROWS_A = 64
_K = 16
_ROWS_C = 632


def _topk_body(a_ref, b_ref, sq_ref, rows_ref, o_ref):
    R = a_ref.shape[0]
    W = b_ref.shape[1]
    def r16(v):
        return v.astype(jnp.bfloat16).astype(jnp.float32)

    xi = r16(a_ref[:, 0:1])
    yi = r16(a_ref[:, 1:2])
    zi = r16(a_ref[:, 2:3])
    xj = r16(b_ref[0:1, :])
    yj = r16(b_ref[1:2, :])
    zj = r16(b_ref[2:3, :])
    sqj = b_ref[3:4, :]
    term = xi * xj + yi * yj + zi * zj
    d = jnp.maximum((sq_ref[...] + sqj) - 2.0 * term, 0.0)
    rows = rows_ref[...]
    cols = lax.broadcasted_iota(jnp.int32, (R, W), 1)
    inf = jnp.float32(jnp.inf)
    d = jnp.where(cols == rows, inf, d)
    m = jnp.min(d, axis=1, keepdims=True)
    for t in range(_K):
        idx = jnp.min(jnp.where(d <= m, cols, W), axis=1, keepdims=True)
        o_ref[:, t:t + 1] = idx
        d = jnp.where(cols == idx, inf, d)
        if t < _K - 1:
            m = jnp.min(d, axis=1, keepdims=True)


def _feat_body(g_ref, nrm_ref, o_ref):
    k = jnp.float32(_K)
    rr = nrm_ref.shape[0]
    nb = g_ref[...].reshape(rr, _K, 16)
    nx, ny, nz = nb[:, :, 0], nb[:, :, 1], nb[:, :, 2]
    mx = jnp.sum(nx, axis=1, keepdims=True) / k
    my = jnp.sum(ny, axis=1, keepdims=True) / k
    mz = jnp.sum(nz, axis=1, keepdims=True) / k
    cx, cy, cz = nx - mx, ny - my, nz - mz
    denom = jnp.float32(_K - 1)
    a00 = jnp.sum(cx * cx, axis=1, keepdims=True) / denom
    a11 = jnp.sum(cy * cy, axis=1, keepdims=True) / denom
    a22 = jnp.sum(cz * cz, axis=1, keepdims=True) / denom
    a01 = jnp.sum(cx * cy, axis=1, keepdims=True) / denom
    a02 = jnp.sum(cx * cz, axis=1, keepdims=True) / denom
    a12 = jnp.sum(cy * cz, axis=1, keepdims=True) / denom

    q = (a00 + a11 + a22) / 3.0
    p1 = a01 * a01 + a02 * a02 + a12 * a12
    p2 = (a00 - q) ** 2 + (a11 - q) ** 2 + (a22 - q) ** 2 + 2.0 * p1
    p = jnp.sqrt(jnp.maximum(p2, 0.0) / 6.0)
    ps = jnp.maximum(p, jnp.float32(1e-20))
    b00, b11, b22 = (a00 - q) / ps, (a11 - q) / ps, (a22 - q) / ps
    b01, b02, b12 = a01 / ps, a02 / ps, a12 / ps
    detb = (b00 * (b11 * b22 - b12 * b12)
            - b01 * (b01 * b22 - b12 * b02)
            + b02 * (b01 * b12 - b11 * b02))
    r = jnp.clip(detb * 0.5, -1.0, 1.0)
    mu0 = jnp.full_like(r, 2.0)
    mu2 = jnp.full_like(r, -2.0)
    for _ in range(20):
        g = mu0 * mu0 * mu0 - 3.0 * mu0 - 2.0 * r
        gp = jnp.maximum(3.0 * mu0 * mu0 - 3.0, jnp.float32(1e-12))
        mu0 = jnp.clip(mu0 - g / gp, 1.0, 2.0)
        g = mu2 * mu2 * mu2 - 3.0 * mu2 - 2.0 * r
        gp = jnp.maximum(3.0 * mu2 * mu2 - 3.0, jnp.float32(1e-12))
        mu2 = jnp.clip(mu2 - g / gp, -2.0, -1.0)
    w0 = q + p * mu0
    w2 = q + p * mu2
    c00, c11, c22 = a00 - w2, a11 - w2, a22 - w2
    v1x = a01 * a12 - a02 * c11
    v1y = a02 * a01 - c00 * a12
    v1z = c00 * c11 - a01 * a01
    v2x = a01 * c22 - a02 * a12
    v2y = a02 * a02 - c00 * c22
    v2z = c00 * a12 - a01 * a02
    v3x = c11 * c22 - a12 * a12
    v3y = a12 * a02 - a01 * c22
    v3z = a01 * a12 - c11 * a02
    n1 = v1x * v1x + v1y * v1y + v1z * v1z
    n2 = v2x * v2x + v2y * v2y + v2z * v2z
    n3 = v3x * v3x + v3y * v3y + v3z * v3z
    use1 = jnp.logical_and(n1 >= n2, n1 >= n3)
    use2 = n2 >= n3
    vx = jnp.where(use1, v1x, jnp.where(use2, v2x, v3x))
    vy = jnp.where(use1, v1y, jnp.where(use2, v2y, v3y))
    vz = jnp.where(use1, v1z, jnp.where(use2, v2z, v3z))
    nv = jnp.sqrt(vx * vx + vy * vy + vz * vz)
    nv = jnp.maximum(nv, jnp.float32(1e-20))
    vx, vy, vz = vx / nv, vy / nv, vz / nv

    nrm = nrm_ref[...]
    n0, n1c, n2c = nrm[:, 0:1], nrm[:, 1:2], nrm[:, 2:3]
    dot = vx * n0 + vy * n1c + vz * n2c
    s = jnp.sign(dot)
    fx, fy, fz = vx * s, vy * s, vz * s
    consistency = jnp.abs(dot)
    total_var = jnp.maximum(a00 + a11 + a22, jnp.float32(1e-8))
    curvature = w2 / total_var
    perp = jnp.abs(cx * fx + cy * fy + cz * fz)
    rough = jnp.sum(perp, axis=1, keepdims=True) / k
    o_ref[...] = jnp.concatenate([consistency, nrm, curvature, rough], axis=1)


def _topk_call(xyz_blk, b_mat, sq_blk, rows_blk):
    rloc = xyz_blk.shape[0]
    npad = b_mat.shape[1]
    grid = rloc // _ROWS_A
    return pl.pallas_call(
        _topk_body,
        grid=(grid,),
        in_specs=[
            pl.BlockSpec((_ROWS_A, 3), lambda i: (i, 0)),
            pl.BlockSpec((8, npad), lambda i: (0, 0)),
            pl.BlockSpec((_ROWS_A, 1), lambda i: (i, 0)),
            pl.BlockSpec((_ROWS_A, 1), lambda i: (i, 0)),
        ],
        out_specs=pl.BlockSpec((_ROWS_A, _K), lambda i: (i, 0)),
        out_shape=jax.ShapeDtypeStruct((rloc, _K), jnp.int32),
    )(xyz_blk, b_mat, sq_blk, rows_blk)


def _sc_gather(table, idx_flat):
    info = plsc.get_sparse_core_info()
    nw = info.num_cores * info.num_subcores
    b = idx_flat.shape[0]
    quantum = nw * 128
    b_pad = ((b + quantum - 1) // quantum) * quantum
    idx_flat = jnp.pad(idx_flat, (0, b_pad - b))
    chunks = b_pad // quantum
    idx3 = idx_flat.reshape(nw, chunks, 128)
    mesh = plsc.VectorSubcoreMesh(core_axis_name="c", subcore_axis_name="s")

    @functools.partial(
        pl.kernel, mesh=mesh,
        compiler_params=pltpu.CompilerParams(use_tc_tiling_on_sc=False),
        out_type=jax.ShapeDtypeStruct((nw, chunks, 128, 16), jnp.float32),
        scratch_types=[
            pltpu.VMEM((chunks, 128), jnp.int32),
            pltpu.VMEM((128, 16), jnp.float32),
            pltpu.SemaphoreType.DMA,
        ],
    )
    def gather_k(table_hbm, idx_hbm, out_hbm, idx_v, rows_v, sem):
        wid = lax.axis_index("s") * info.num_cores + lax.axis_index("c")
        pltpu.sync_copy(idx_hbm.at[wid], idx_v)

        def body(j, carry):
            pltpu.async_copy(table_hbm.at[idx_v.at[j]], rows_v, sem).wait()
            pltpu.sync_copy(rows_v, out_hbm.at[wid, j])
            return carry

        lax.fori_loop(0, chunks, body, 0)

    out = gather_k(table, idx3)
    return out.reshape(b_pad, 16)


def _feat_call(g_rows, normals):
    nloc = normals.shape[0]
    grid = nloc // _ROWS_C
    return pl.pallas_call(
        _feat_body,
        grid=(grid,),
        in_specs=[
            pl.BlockSpec((_ROWS_C * _K, 16), lambda i: (i, 0)),
            pl.BlockSpec((_ROWS_C, 3), lambda i: (i, 0)),
        ],
        out_specs=pl.BlockSpec((_ROWS_C, 6), lambda i: (i, 0)),
        out_shape=jax.ShapeDtypeStruct((nloc, 6), jnp.float32),
    )(g_rows, normals)


def kernel(xyz, normals):
    n = xyz.shape[0]
    ctx = jax.sharding.get_abstract_mesh()
    if not ctx.empty and ctx.axis_names == ("x",):
        mesh = ctx
        ndev = ctx.shape["x"]
    else:
        devs = jax.devices()
        ndev = len(devs)
        mesh = Mesh(np.array(devs), ("x",))
    repl = P(None, None)
    shard = P("x", None)

    quantum = ndev * _ROWS_A
    npad = ((n + quantum - 1) // quantum) * quantum
    xyz_p = jnp.pad(xyz, ((0, npad - n), (0, 0)), constant_values=1e9)
    sq = jnp.sum(xyz_p * xyz_p, axis=1, keepdims=True)
    zeros = jnp.zeros((npad, 4), jnp.float32)
    b_mat = jnp.concatenate([xyz_p.T, sq.T, zeros.T], axis=0)
    rows_g = jnp.arange(npad, dtype=jnp.int32)[:, None]

    topk = shard_map(_topk_call, mesh=mesh,
                     in_specs=(shard, repl, shard, shard),
                     out_specs=shard, check_rep=False)
    idx = topk(xyz_p, b_mat, sq, rows_g)

    table = jnp.pad(xyz_p, ((0, 0), (0, 13)))
    gather = shard_map(_sc_gather, mesh=mesh,
                       in_specs=(repl, P("x")), out_specs=shard,
                       check_rep=False)
    g_rows = gather(table, idx.reshape(-1))
    normals_p = jnp.pad(normals, ((0, npad - n), (0, 0)))

    feats = shard_map(_feat_call, mesh=mesh,
                      in_specs=(shard, shard), out_specs=shard,
                      check_rep=False)
    return feats(g_rows, normals_p)[:n]


# speedup vs baseline: 22.5380x; 1.0395x over previous
import functools

import jax
import jax.numpy as jnp
import numpy as np
from jax import lax
from jax.experimental import pallas as pl
from jax.experimental.pallas import tpu as pltpu
from jax.experimental.pallas import tpu_sc as plsc
from jax.sharding import Mesh, PartitionSpec as P
from jax.experimental.shard_map import shard_map

_ROWS_A = 64
_K = 16
_ROWS_C = 632


def _topk_body(a_ref, b_ref, sq_ref, rows_ref, o_ref):
    R = a_ref.shape[0]
    W = b_ref.shape[1]
    a16 = a_ref[...].astype(jnp.bfloat16)
    b16 = b_ref[0:3, :].astype(jnp.bfloat16)
    term = jnp.dot(a16, b16, preferred_element_type=jnp.float32)
    sqj = b_ref[3:4, :]
    d = jnp.maximum((sq_ref[...] + sqj) - 2.0 * term, 0.0)
    rows = rows_ref[...]
    cols = lax.broadcasted_iota(jnp.int32, (R, W), 1)
    inf = jnp.float32(jnp.inf)
    d = jnp.where(cols == rows, inf, d)
    m = jnp.min(d, axis=1, keepdims=True)
    for t in range(_K):
        idx = jnp.min(jnp.where(d <= m, cols, W), axis=1, keepdims=True)
        o_ref[:, t:t + 1] = idx
        d = jnp.where(cols == idx, inf, d)
        if t < _K - 1:
            m = jnp.min(d, axis=1, keepdims=True)


def _feat_body(g_ref, nrm_ref, o_ref):
    k = jnp.float32(_K)
    rr = nrm_ref.shape[0]
    nb = g_ref[...].reshape(rr, _K, 16)
    nx, ny, nz = nb[:, :, 0], nb[:, :, 1], nb[:, :, 2]
    mx = jnp.sum(nx, axis=1, keepdims=True) / k
    my = jnp.sum(ny, axis=1, keepdims=True) / k
    mz = jnp.sum(nz, axis=1, keepdims=True) / k
    cx, cy, cz = nx - mx, ny - my, nz - mz
    denom = jnp.float32(_K - 1)
    a00 = jnp.sum(cx * cx, axis=1, keepdims=True) / denom
    a11 = jnp.sum(cy * cy, axis=1, keepdims=True) / denom
    a22 = jnp.sum(cz * cz, axis=1, keepdims=True) / denom
    a01 = jnp.sum(cx * cy, axis=1, keepdims=True) / denom
    a02 = jnp.sum(cx * cz, axis=1, keepdims=True) / denom
    a12 = jnp.sum(cy * cz, axis=1, keepdims=True) / denom

    q = (a00 + a11 + a22) / 3.0
    p1 = a01 * a01 + a02 * a02 + a12 * a12
    p2 = (a00 - q) ** 2 + (a11 - q) ** 2 + (a22 - q) ** 2 + 2.0 * p1
    p = jnp.sqrt(jnp.maximum(p2, 0.0) / 6.0)
    ps = jnp.maximum(p, jnp.float32(1e-20))
    b00, b11, b22 = (a00 - q) / ps, (a11 - q) / ps, (a22 - q) / ps
    b01, b02, b12 = a01 / ps, a02 / ps, a12 / ps
    detb = (b00 * (b11 * b22 - b12 * b12)
            - b01 * (b01 * b22 - b12 * b02)
            + b02 * (b01 * b12 - b11 * b02))
    r = jnp.clip(detb * 0.5, -1.0, 1.0)
    mu0 = jnp.full_like(r, 2.0)
    mu2 = jnp.full_like(r, -2.0)
    for _ in range(20):
        g = mu0 * mu0 * mu0 - 3.0 * mu0 - 2.0 * r
        gp = jnp.maximum(3.0 * mu0 * mu0 - 3.0, jnp.float32(1e-12))
        mu0 = jnp.clip(mu0 - g / gp, 1.0, 2.0)
        g = mu2 * mu2 * mu2 - 3.0 * mu2 - 2.0 * r
        gp = jnp.maximum(3.0 * mu2 * mu2 - 3.0, jnp.float32(1e-12))
        mu2 = jnp.clip(mu2 - g / gp, -2.0, -1.0)
    w0 = q + p * mu0
    w2 = q + p * mu2
    c00, c11, c22 = a00 - w2, a11 - w2, a22 - w2
    v1x = a01 * a12 - a02 * c11
    v1y = a02 * a01 - c00 * a12
    v1z = c00 * c11 - a01 * a01
    v2x = a01 * c22 - a02 * a12
    v2y = a02 * a02 - c00 * c22
    v2z = c00 * a12 - a01 * a02
    v3x = c11 * c22 - a12 * a12
    v3y = a12 * a02 - a01 * c22
    v3z = a01 * a12 - c11 * a02
    n1 = v1x * v1x + v1y * v1y + v1z * v1z
    n2 = v2x * v2x + v2y * v2y + v2z * v2z
    n3 = v3x * v3x + v3y * v3y + v3z * v3z
    use1 = jnp.logical_and(n1 >= n2, n1 >= n3)
    use2 = n2 >= n3
    vx = jnp.where(use1, v1x, jnp.where(use2, v2x, v3x))
    vy = jnp.where(use1, v1y, jnp.where(use2, v2y, v3y))
    vz = jnp.where(use1, v1z, jnp.where(use2, v2z, v3z))
    nv = jnp.sqrt(vx * vx + vy * vy + vz * vz)
    nv = jnp.maximum(nv, jnp.float32(1e-20))
    vx, vy, vz = vx / nv, vy / nv, vz / nv

    nrm = nrm_ref[...]
    n0, n1c, n2c = nrm[:, 0:1], nrm[:, 1:2], nrm[:, 2:3]
    dot = vx * n0 + vy * n1c + vz * n2c
    s = jnp.sign(dot)
    fx, fy, fz = vx * s, vy * s, vz * s
    consistency = jnp.abs(dot)
    total_var = jnp.maximum(a00 + a11 + a22, jnp.float32(1e-8))
    curvature = w2 / total_var
    perp = jnp.abs(cx * fx + cy * fy + cz * fz)
    rough = jnp.sum(perp, axis=1, keepdims=True) / k
    o_ref[...] = jnp.concatenate([consistency, nrm, curvature, rough], axis=1)


def _topk_call(xyz_blk, b_mat, sq_blk, rows_blk):
    rloc = xyz_blk.shape[0]
    npad = b_mat.shape[1]
    grid = rloc // _ROWS_A
    return pl.pallas_call(
        _topk_body,
        grid=(grid,),
        in_specs=[
            pl.BlockSpec((_ROWS_A, 3), lambda i: (i, 0)),
            pl.BlockSpec((8, npad), lambda i: (0, 0)),
            pl.BlockSpec((_ROWS_A, 1), lambda i: (i, 0)),
            pl.BlockSpec((_ROWS_A, 1), lambda i: (i, 0)),
        ],
        out_specs=pl.BlockSpec((_ROWS_A, _K), lambda i: (i, 0)),
        out_shape=jax.ShapeDtypeStruct((rloc, _K), jnp.int32),
    )(xyz_blk, b_mat, sq_blk, rows_blk)


def _sc_gather(table, idx_flat):
    info = plsc.get_sparse_core_info()
    nw = info.num_cores * info.num_subcores
    b = idx_flat.shape[0]
    quantum = nw * 128
    b_pad = ((b + quantum - 1) // quantum) * quantum
    idx_flat = jnp.pad(idx_flat, (0, b_pad - b))
    chunks = b_pad // quantum
    idx3 = idx_flat.reshape(nw, chunks, 128)
    mesh = plsc.VectorSubcoreMesh(core_axis_name="c", subcore_axis_name="s")

    @functools.partial(
        pl.kernel, mesh=mesh,
        compiler_params=pltpu.CompilerParams(use_tc_tiling_on_sc=False),
        out_type=jax.ShapeDtypeStruct((nw, chunks, 128, 16), jnp.float32),
        scratch_types=[
            pltpu.VMEM((chunks, 128), jnp.int32),
            pltpu.VMEM((128, 16), jnp.float32),
            pltpu.SemaphoreType.DMA,
        ],
    )
    def gather_k(table_hbm, idx_hbm, out_hbm, idx_v, rows_v, sem):
        wid = lax.axis_index("s") * info.num_cores + lax.axis_index("c")
        pltpu.sync_copy(idx_hbm.at[wid], idx_v)

        def body(j, carry):
            pltpu.async_copy(table_hbm.at[idx_v.at[j]], rows_v, sem).wait()
            pltpu.sync_copy(rows_v, out_hbm.at[wid, j])
            return carry

        lax.fori_loop(0, chunks, body, 0)

    out = gather_k(table, idx3)
    return out.reshape(b_pad, 16)


def _feat_call(g_rows, normals):
    nloc = normals.shape[0]
    grid = nloc // _ROWS_C
    return pl.pallas_call(
        _feat_body,
        grid=(grid,),
        in_specs=[
            pl.BlockSpec((_ROWS_C * _K, 16), lambda i: (i, 0)),
            pl.BlockSpec((_ROWS_C, 3), lambda i: (i, 0)),
        ],
        out_specs=pl.BlockSpec((_ROWS_C, 6), lambda i: (i, 0)),
        out_shape=jax.ShapeDtypeStruct((nloc, 6), jnp.float32),
    )(g_rows, normals)


def kernel(xyz, normals):
    n = xyz.shape[0]
    ctx = jax.sharding.get_abstract_mesh()
    if not ctx.empty and ctx.axis_names == ("x",):
        mesh = ctx
        ndev = ctx.shape["x"]
    else:
        devs = jax.devices()
        ndev = len(devs)
        mesh = Mesh(np.array(devs), ("x",))
    repl = P(None, None)
    shard = P("x", None)

    quantum = ndev * _ROWS_A
    npad = ((n + quantum - 1) // quantum) * quantum
    xyz_p = jnp.pad(xyz, ((0, npad - n), (0, 0)), constant_values=1e9)
    sq = jnp.sum(xyz_p * xyz_p, axis=1, keepdims=True)
    zeros = jnp.zeros((npad, 4), jnp.float32)
    b_mat = jnp.concatenate([xyz_p.T, sq.T, zeros.T], axis=0)
    rows_g = jnp.arange(npad, dtype=jnp.int32)[:, None]

    topk = shard_map(_topk_call, mesh=mesh,
                     in_specs=(shard, repl, shard, shard),
                     out_specs=shard, check_rep=False)
    idx = topk(xyz_p, b_mat, sq, rows_g)

    table = jnp.pad(xyz_p, ((0, 0), (0, 13)))
    gather = shard_map(_sc_gather, mesh=mesh,
                       in_specs=(repl, P("x")), out_specs=shard,
                       check_rep=False)
    g_rows = gather(table, idx.reshape(-1))
    normals_p = jnp.pad(normals, ((0, npad - n), (0, 0)))

    feats = shard_map(_feat_call, mesh=mesh,
                      in_specs=(shard, shard), out_specs=shard,
                      check_rep=False)
    return feats(g_rows, normals_p)[:n]
